# baseline (speedup 1.0000x reference)
import functools

import jax
import jax.numpy as jnp
from jax import lax
from jax.experimental import pallas as pl
from jax.experimental.pallas import tpu as pltpu
from jax.experimental.pallas import tpu_sc as plsc

DIM = 64
EPS = 1e-05
NC, NS = 2, 16
NW = NC * NS
CHUNK = 128
L = 16
NV = DIM // L
UNROLL = 16


def _rsqrt(a):
    i = lax.bitcast_convert_type(a, jnp.int32)
    i = jnp.int32(0x5F3759DF) - lax.shift_right_arithmetic(i, 1)
    y = lax.bitcast_convert_type(i, jnp.float32)
    xh = a * 0.5
    y = y * (1.5 - xh * y * y)
    y = y * (1.5 - xh * y * y)
    y = y * (1.5 - xh * y * y)
    return y


def _make_call(nchunk):
    rows_per_w = nchunk * CHUNK
    total = NW * rows_per_w
    mesh = plsc.VectorSubcoreMesh(core_axis_name="c", subcore_axis_name="s")

    @functools.partial(
        pl.kernel,
        mesh=mesh,
        compiler_params=pltpu.CompilerParams(needs_layout_passes=False),
        out_type=jax.ShapeDtypeStruct((total // 2, 2 * DIM), jnp.float32),
        scratch_types=[
            pltpu.VMEM((nchunk, CHUNK), jnp.int32),
            pltpu.VMEM((nchunk, CHUNK), jnp.int32),
            pltpu.VMEM((CHUNK, 2 * DIM), jnp.float32),
            pltpu.VMEM((CHUNK, 2 * DIM), jnp.float32),
            pltpu.VMEM((CHUNK // 2, 2 * DIM), jnp.float32),
            pltpu.VMEM((CHUNK // 2, 2 * DIM), jnp.float32),
            pltpu.VMEM((2, DIM), jnp.float32),
            pltpu.SemaphoreType.DMA,
            pltpu.SemaphoreType.DMA,
            pltpu.SemaphoreType.DMA,
            pltpu.SemaphoreType.DMA,
        ],
    )
    def call(idx_hbm, par_hbm, table_hbm, scale_hbm, bias_hbm, out_hbm,
             idx_v, par_v, xb0, xb1, ob0, ob1, sb_v, sg0, sg1, so0, so1):
        cid = lax.axis_index("c")
        sid = lax.axis_index("s")
        wid = sid * NC + cid

        pltpu.sync_copy(idx_hbm.at[wid], idx_v)
        pltpu.sync_copy(par_hbm.at[wid], par_v)
        pltpu.sync_copy(scale_hbm, sb_v.at[0])
        pltpu.sync_copy(bias_hbm, sb_v.at[1])
        sv = [sb_v[0, pl.ds(L * k, L)] for k in range(NV)]
        bv = [sb_v[1, pl.ds(L * k, L)] for k in range(NV)]
        out_base = wid * (rows_per_w // 2)

        def ln_rows(xb, ob, c, i):
            pvec = par_v[c, pl.ds(UNROLL * i, UNROLL)]
            xs = []
            ss = []
            qs = []
            for j in range(UNROLL):
                r = i * UNROLL + j
                pj = pvec[j]
                x = [xb[r, pl.ds(pj + L * k, L)] for k in range(NV)]
                s = (x[0] + x[1]) + (x[2] + x[3])
                q = (x[0] * x[0] + x[1] * x[1]) + (x[2] * x[2] + x[3] * x[3])
                xs.append(x)
                ss.append(jnp.sum(s))
                qs.append(jnp.sum(q))
            coefs = []
            for j in range(UNROLL):
                mean = ss[j] * (1.0 / DIM)
                var = qs[j] * (1.0 / DIM) - mean * mean
                rinv = _rsqrt(var + EPS)
                mr = mean * rinv
                coefs.append((lax.broadcast_in_dim(rinv, (L,), ()),
                              lax.broadcast_in_dim(mr, (L,), ())))
            for j in range(UNROLL):
                pr = i * (UNROLL // 2) + j // 2
                half = (j % 2) * DIM
                rsj, mrj = coefs[j]
                for k in range(NV):
                    o = xs[j][k] * (sv[k] * rsj) + (bv[k] - sv[k] * mrj)
                    ob[pr, pl.ds(half + L * k, L)] = o

        def compute(xb, ob, c):
            lax.fori_loop(0, CHUNK // UNROLL,
                          lambda i, cc: (ln_rows(xb, ob, cc, i), cc)[1], c)

        opr = CHUNK // 2

        def pair_body(p, carry):
            c0 = 2 * p
            c1 = c0 + 1
            pltpu.async_copy(table_hbm.at[idx_v.at[c1]], xb1, sg1)
            pltpu.make_async_copy(table_hbm.at[idx_v.at[c0]], xb0, sg0).wait()

            @pl.when(p > 0)
            def _():
                pltpu.make_async_copy(
                    ob0, out_hbm.at[pl.ds(out_base + (c0 - 2) * opr, opr)],
                    so0).wait()
            compute(xb0, ob0, c0)
            pltpu.async_copy(
                ob0, out_hbm.at[pl.ds(out_base + c0 * opr, opr)], so0)

            @pl.when(c1 + 1 < nchunk)
            def _():
                pltpu.async_copy(table_hbm.at[idx_v.at[c1 + 1]], xb0, sg0)

            pltpu.make_async_copy(table_hbm.at[idx_v.at[c1]], xb1, sg1).wait()

            @pl.when(p > 0)
            def _():
                pltpu.make_async_copy(
                    ob1, out_hbm.at[pl.ds(out_base + (c1 - 2) * opr, opr)],
                    so1).wait()
            compute(xb1, ob1, c1)
            pltpu.async_copy(
                ob1, out_hbm.at[pl.ds(out_base + c1 * opr, opr)], so1)
            return carry

        pltpu.async_copy(table_hbm.at[idx_v.at[0]], xb0, sg0)
        lax.fori_loop(0, nchunk // 2, pair_body, 0)
        pltpu.make_async_copy(
            ob0, out_hbm.at[pl.ds(out_base + (nchunk - 2) * opr, opr)],
            so0).wait()
        pltpu.make_async_copy(
            ob1, out_hbm.at[pl.ds(out_base + (nchunk - 1) * opr, opr)],
            so1).wait()

    return call


_CALLS = {}


def kernel(inputs, emb_weight, ln_scale, ln_bias):
    b, s = inputs.shape
    total = b * s
    assert total % (NW * 2 * CHUNK) == 0
    nchunk = total // (NW * CHUNK)
    if nchunk not in _CALLS:
        _CALLS[nchunk] = _make_call(nchunk)
    idx = inputs.astype(jnp.int32).reshape(NW, nchunk, CHUNK)
    idxd = lax.shift_right_logical(idx, 1)
    par = lax.shift_left(jnp.bitwise_and(idx, 1), 6)
    table2 = emb_weight.reshape(emb_weight.shape[0] // 2, 2 * DIM)
    out = _CALLS[nchunk](idxd, par, table2, ln_scale, ln_bias)
    return out.reshape(b, s, DIM)


# speedup vs baseline: 1.6710x; 1.6710x over previous
import functools

import jax
import jax.numpy as jnp
from jax import lax
from jax.experimental import pallas as pl
from jax.experimental.pallas import tpu as pltpu
from jax.experimental.pallas import tpu_sc as plsc

DIM = 64
EPS = 1e-05
NC, NS = 2, 16
NW = NC * NS
CHUNK = 128
L = 16
NV = DIM // L
UNROLL = 16
CW = CHUNK * DIM


def _rsqrt(a):
    i = lax.bitcast_convert_type(a, jnp.int32)
    i = jnp.int32(0x5F3759DF) - lax.shift_right_arithmetic(i, 1)
    y = lax.bitcast_convert_type(i, jnp.float32)
    xh = a * 0.5
    y = y * (1.5 - xh * y * y)
    y = y * (1.5 - xh * y * y)
    y = y * (1.5 - xh * y * y)
    return y


def _make_call(nchunk):
    rows_per_w = nchunk * CHUNK
    total = NW * rows_per_w
    mesh = plsc.VectorSubcoreMesh(core_axis_name="c", subcore_axis_name="s")

    @functools.partial(
        pl.kernel,
        mesh=mesh,
        compiler_params=pltpu.CompilerParams(needs_layout_passes=False),
        out_type=jax.ShapeDtypeStruct((total * DIM,), jnp.float32),
        scratch_types=[
            pltpu.VMEM((nchunk, CHUNK), jnp.int32),
            pltpu.VMEM((CHUNK, DIM), jnp.float32),
            pltpu.VMEM((CHUNK, DIM), jnp.float32),
            pltpu.VMEM((CW,), jnp.float32),
            pltpu.VMEM((CW,), jnp.float32),
            pltpu.VMEM((2, DIM), jnp.float32),
            pltpu.SemaphoreType.DMA,
            pltpu.SemaphoreType.DMA,
            pltpu.SemaphoreType.DMA,
            pltpu.SemaphoreType.DMA,
        ],
    )
    def call(idx_hbm, table_hbm, scale_hbm, bias_hbm, out_hbm,
             idx_v, xb0, xb1, ob0, ob1, sb_v, sg0, sg1, so0, so1):
        cid = lax.axis_index("c")
        sid = lax.axis_index("s")
        wid = sid * NC + cid

        pltpu.sync_copy(idx_hbm.at[wid], idx_v)
        pltpu.sync_copy(scale_hbm, sb_v.at[0])
        pltpu.sync_copy(bias_hbm, sb_v.at[1])
        sv = [sb_v[0, pl.ds(L * k, L)] for k in range(NV)]
        bv = [sb_v[1, pl.ds(L * k, L)] for k in range(NV)]
        out_base = wid * rows_per_w * DIM

        def issue_gathers(c, xb, sem):
            def gi(i, carry):
                iv = idx_v[c, pl.ds(UNROLL * i, UNROLL)]
                for j in range(UNROLL):
                    dst = xb.at[UNROLL * i + j]
                    pltpu.async_copy(table_hbm.at[iv[j]], dst, sem)
                return carry
            lax.fori_loop(0, CHUNK // UNROLL, gi, 0)

        def drain_gathers(xb, sem):
            pltpu.make_async_copy(
                table_hbm.at[pl.ds(0, CHUNK)], xb, sem).wait()

        def ln_rows(xb, ob, i):
            xs = []
            ss = []
            qs = []
            for j in range(UNROLL):
                r = UNROLL * i + j
                base = CW // (CHUNK // UNROLL) * i + DIM * j
                x = [xb[r, pl.ds(L * k, L)] for k in range(NV)]
                s = (x[0] + x[1]) + (x[2] + x[3])
                q = (x[0] * x[0] + x[1] * x[1]) + (x[2] * x[2] + x[3] * x[3])
                xs.append(x)
                ss.append(jnp.sum(s))
                qs.append(jnp.sum(q))
            coefs = []
            for j in range(UNROLL):
                mean = ss[j] * (1.0 / DIM)
                var = qs[j] * (1.0 / DIM) - mean * mean
                rinv = _rsqrt(var + EPS)
                mr = mean * rinv
                coefs.append((lax.broadcast_in_dim(rinv, (L,), ()),
                              lax.broadcast_in_dim(mr, (L,), ())))
            for j in range(UNROLL):
                base = CW // (CHUNK // UNROLL) * i + DIM * j
                rsj, mrj = coefs[j]
                for k in range(NV):
                    o = xs[j][k] * (sv[k] * rsj) + (bv[k] - sv[k] * mrj)
                    ob[pl.ds(base + L * k, L)] = o

        def compute(xb, ob):
            lax.fori_loop(0, CHUNK // UNROLL,
                          lambda i, cc: (ln_rows(xb, ob, i), cc)[1], 0)

        def pair_body(p, carry):
            c0 = 2 * p
            c1 = c0 + 1
            issue_gathers(c1, xb1, sg1)
            drain_gathers(xb0, sg0)

            @pl.when(p > 0)
            def _():
                pltpu.make_async_copy(
                    ob0, out_hbm.at[pl.ds(out_base + (c0 - 2) * CW, CW)],
                    so0).wait()
            compute(xb0, ob0)
            pltpu.async_copy(
                ob0, out_hbm.at[pl.ds(out_base + c0 * CW, CW)], so0)

            @pl.when(c1 + 1 < nchunk)
            def _():
                issue_gathers(c1 + 1, xb0, sg0)

            drain_gathers(xb1, sg1)

            @pl.when(p > 0)
            def _():
                pltpu.make_async_copy(
                    ob1, out_hbm.at[pl.ds(out_base + (c1 - 2) * CW, CW)],
                    so1).wait()
            compute(xb1, ob1)
            pltpu.async_copy(
                ob1, out_hbm.at[pl.ds(out_base + c1 * CW, CW)], so1)
            return carry

        issue_gathers(0, xb0, sg0)
        lax.fori_loop(0, nchunk // 2, pair_body, 0)
        pltpu.make_async_copy(
            ob0, out_hbm.at[pl.ds(out_base + (nchunk - 2) * CW, CW)],
            so0).wait()
        pltpu.make_async_copy(
            ob1, out_hbm.at[pl.ds(out_base + (nchunk - 1) * CW, CW)],
            so1).wait()

    return call


_CALLS = {}


def kernel(inputs, emb_weight, ln_scale, ln_bias):
    b, s = inputs.shape
    total = b * s
    assert total % (NW * 2 * CHUNK) == 0
    nchunk = total // (NW * CHUNK)
    if nchunk not in _CALLS:
        _CALLS[nchunk] = _make_call(nchunk)
    idx = inputs.astype(jnp.int32).reshape(NW, nchunk, CHUNK)
    out = _CALLS[nchunk](idx, emb_weight, ln_scale, ln_bias)
    return out.reshape(b, s, DIM)
